# Initial kernel scaffold; baseline (speedup 1.0000x reference)
import jax
import jax.numpy as jnp
from jax.experimental import pallas as pl


def kernel(x):
    raise NotImplementedError("write your pallas kernel here")


# speedup vs baseline: 6.7114x; 6.7114x over previous
import jax
import jax.numpy as jnp
from jax.experimental import pallas as pl
from jax.experimental.pallas import tpu as pltpu

MAX_OBJ = 100
IOU_THRES = 0.45
SCORE_THRES = 0.25
NC = 80
MAX_WH = 640.0
N_BOXES = 20000
NPAD = 20480
NEG = -1.0e30
BIGI = 1 << 30


def _nms_kernel(xt_ref, dets_ref, f_ref):
    conf = xt_ref[4:5, :]
    cls = xt_ref[5:85, :]
    scores = conf * cls
    cs = jnp.max(scores, axis=0, keepdims=True)
    csub = jax.lax.broadcasted_iota(jnp.int32, (NC, NPAD), 0).astype(jnp.float32)
    ci = (NC - 1) - jnp.max(
        jnp.where(scores == cs, (NC - 1) - csub, -1.0), axis=0, keepdims=True
    )
    bx1 = xt_ref[0:1, :] - xt_ref[2:3, :] * 0.5
    by1 = xt_ref[1:2, :] - xt_ref[3:4, :] * 0.5
    bx2 = xt_ref[0:1, :] + xt_ref[2:3, :] * 0.5
    by2 = xt_ref[1:2, :] + xt_ref[3:4, :] * 0.5
    off = ci * MAX_WH
    ox1 = bx1 + off
    oy1 = by1 + off
    ox2 = bx2 + off
    oy2 = by2 + off
    area = (ox2 - ox1) * (oy2 - oy1)
    avail = jnp.where(cs > SCORE_THRES, cs, NEG)
    f_ref[0:1, :] = avail
    f_ref[1:2, :] = ox1
    f_ref[2:3, :] = oy1
    f_ref[3:4, :] = ox2
    f_ref[4:5, :] = oy2
    f_ref[5:6, :] = bx1
    f_ref[6:7, :] = by1
    f_ref[7:8, :] = bx2
    f_ref[8:9, :] = by2
    f_ref[9:10, :] = ci
    f_ref[10:11, :] = area

    idxv = jax.lax.broadcasted_iota(jnp.int32, (1, NPAD), 1)
    lane = jax.lax.broadcasted_iota(jnp.int32, (1, 128), 1)

    def body(it, _):
        av = f_ref[0:1, :]
        m = jnp.max(av)
        ok = m > 0.0
        wi = jnp.min(jnp.where(av == m, idxv, BIGI))
        sel = idxv == wi

        def pick(r):
            return jnp.max(jnp.where(sel, f_ref[r:r + 1, :], NEG))

        wx1, wy1, wx2, wy2 = pick(1), pick(2), pick(3), pick(4)
        warea = pick(10)
        ix1 = jnp.maximum(wx1, f_ref[1:2, :])
        iy1 = jnp.maximum(wy1, f_ref[2:3, :])
        ix2 = jnp.minimum(wx2, f_ref[3:4, :])
        iy2 = jnp.minimum(wy2, f_ref[4:5, :])
        inter = jnp.maximum(ix2 - ix1, 0.0) * jnp.maximum(iy2 - iy1, 0.0)
        iou = inter / (warea + f_ref[10:11, :] - inter + 1e-9)
        supp = jnp.logical_or(jnp.logical_and(ok, iou > IOU_THRES), sel)
        f_ref[0:1, :] = jnp.where(supp, NEG, av)

        rb1, rb2, rb3, rb4 = pick(5), pick(6), pick(7), pick(8)
        wcls = pick(9)
        row = jnp.where(lane == 0, jnp.where(ok, rb1, 0.0), 0.0)
        row = jnp.where(lane == 1, jnp.where(ok, rb2, 0.0), row)
        row = jnp.where(lane == 2, jnp.where(ok, rb3, 0.0), row)
        row = jnp.where(lane == 3, jnp.where(ok, rb4, 0.0), row)
        row = jnp.where(lane == 4, jnp.where(ok, m, 0.0), row)
        row = jnp.where(lane == 5, jnp.where(ok, wcls, -1.0), row)
        dets_ref[pl.ds(it, 1), :] = row
        return 0

    jax.lax.fori_loop(0, MAX_OBJ, body, 0)


def _run_nms(xt, interpret=False):
    return pl.pallas_call(
        _nms_kernel,
        out_shape=jax.ShapeDtypeStruct((MAX_OBJ, 128), jnp.float32),
        scratch_shapes=[pltpu.VMEM((11, NPAD), jnp.float32)],
        interpret=interpret,
    )(xt)


def kernel(x):
    xp = jnp.pad(x[0], ((0, NPAD - N_BOXES), (0, 0)))
    xt = xp.T
    dets = _run_nms(xt)
    return dets[None, :, :6]


# speedup vs baseline: 15.3524x; 2.2875x over previous
import jax
import jax.numpy as jnp
from jax.experimental import pallas as pl
from jax.experimental.pallas import tpu as pltpu

MAX_OBJ = 100
IOU_THRES = 0.45
SCORE_THRES = 0.25
NC = 80
MAX_WH = 640.0
N_BOXES = 20000
NPAD = 20480
NROW = NPAD // 128
NEG = -1.0e30
BIGI = 1 << 30


def _nms_kernel(xt_ref, dets_ref, f_ref):
    conf = xt_ref[4]
    cls = xt_ref[5:85]
    scores = conf[None] * cls
    cs = jnp.max(scores, axis=0)
    csub = jax.lax.broadcasted_iota(jnp.int32, (NC, NROW, 128), 0).astype(
        jnp.float32)
    ci = (NC - 1) - jnp.max(
        jnp.where(scores == cs[None], (NC - 1) - csub, -1.0), axis=0)
    bx1 = xt_ref[0] - xt_ref[2] * 0.5
    by1 = xt_ref[1] - xt_ref[3] * 0.5
    bx2 = xt_ref[0] + xt_ref[2] * 0.5
    by2 = xt_ref[1] + xt_ref[3] * 0.5
    off = ci * MAX_WH
    ox1 = bx1 + off
    oy1 = by1 + off
    ox2 = bx2 + off
    oy2 = by2 + off
    area = (ox2 - ox1) * (oy2 - oy1)
    f_ref[0] = jnp.where(cs > SCORE_THRES, cs, NEG)
    f_ref[1] = ox1
    f_ref[2] = oy1
    f_ref[3] = ox2
    f_ref[4] = oy2
    f_ref[5] = bx1
    f_ref[6] = by1
    f_ref[7] = bx2
    f_ref[8] = by2
    f_ref[9] = ci
    f_ref[10] = area

    ridx = jax.lax.broadcasted_iota(jnp.int32, (NROW, 128), 0)
    lidx = jax.lax.broadcasted_iota(jnp.int32, (NROW, 128), 1)
    idxv = ridx * 128 + lidx
    lane = jax.lax.broadcasted_iota(jnp.int32, (1, 128), 1)

    def body(it, _):
        av = f_ref[0]
        m = jnp.max(av)
        ok = m > 0.0
        wi = jnp.min(jnp.where(av == m, idxv, BIGI))
        sel = idxv == wi

        def pick(r):
            return jnp.max(jnp.where(sel, f_ref[r], NEG))

        rb1, rb2, rb3, rb4 = pick(5), pick(6), pick(7), pick(8)
        wcls = pick(9)
        woff = wcls * MAX_WH
        wx1 = rb1 + woff
        wy1 = rb2 + woff
        wx2 = rb3 + woff
        wy2 = rb4 + woff
        warea = (wx2 - wx1) * (wy2 - wy1)
        ix1 = jnp.maximum(wx1, f_ref[1])
        iy1 = jnp.maximum(wy1, f_ref[2])
        ix2 = jnp.minimum(wx2, f_ref[3])
        iy2 = jnp.minimum(wy2, f_ref[4])
        inter = jnp.maximum(ix2 - ix1, 0.0) * jnp.maximum(iy2 - iy1, 0.0)
        iou = inter / (warea + f_ref[10] - inter + 1e-9)
        supp = jnp.logical_or(jnp.logical_and(ok, iou > IOU_THRES), sel)
        f_ref[0] = jnp.where(supp, NEG, av)

        row = jnp.where(lane == 0, jnp.where(ok, rb1, 0.0), 0.0)
        row = jnp.where(lane == 1, jnp.where(ok, rb2, 0.0), row)
        row = jnp.where(lane == 2, jnp.where(ok, rb3, 0.0), row)
        row = jnp.where(lane == 3, jnp.where(ok, rb4, 0.0), row)
        row = jnp.where(lane == 4, jnp.where(ok, m, 0.0), row)
        row = jnp.where(lane == 5, jnp.where(ok, wcls, -1.0), row)
        dets_ref[pl.ds(it, 1), :] = row
        return 0

    jax.lax.fori_loop(0, MAX_OBJ, body, 0)


def _run_nms(xt, interpret=False):
    return pl.pallas_call(
        _nms_kernel,
        out_shape=jax.ShapeDtypeStruct((MAX_OBJ, 128), jnp.float32),
        scratch_shapes=[pltpu.VMEM((11, NROW, 128), jnp.float32)],
        interpret=interpret,
    )(xt)


def kernel(x):
    xp = jnp.pad(x[0], ((0, NPAD - N_BOXES), (0, 0)))
    xt = xp.T.reshape(85, NROW, 128)
    dets = _run_nms(xt)
    return dets[None, :, :6]
